# baseline (device time: 2128122 ns/iter reference)
import jax
import jax.numpy as jnp
from jax import lax
from jax.experimental import pallas as pl
from jax.experimental.pallas import tpu as pltpu

NUM_CHUNKS = 16


def kernel(x):
    m_per, n = x.shape
    m_out = 2 * m_per
    half = m_per // 2
    chunk = half // NUM_CHUNKS

    def body(x_ref, out_ref, local_sems,
             x_send_sems, x_recv_sems, y_send_sems, y_recv_sems):
        my_x = lax.axis_index("x")
        my_y = lax.axis_index("y")
        x_nbr = (1 - my_x, my_y)
        y_nbr = (my_x, 1 - my_y)

        barrier_sem = pltpu.get_barrier_semaphore()
        for nbr in (x_nbr, y_nbr):
            pl.semaphore_signal(
                barrier_sem, inc=1,
                device_id=nbr, device_id_type=pl.DeviceIdType.MESH,
            )
        pl.semaphore_wait(barrier_sem, 2)

        lchunk = m_per // NUM_CHUNKS
        local_copies = []
        for c in range(NUM_CHUNKS):
            cp = pltpu.make_async_copy(
                x_ref.at[pl.ds(c * lchunk, lchunk), :],
                out_ref.at[pl.ds(my_x * m_per + c * lchunk, lchunk), :],
                local_sems.at[c],
            )
            cp.start()
            local_copies.append(cp)

        send_base = my_x * m_per + my_y * half
        recv_base = (1 - my_x) * m_per + my_y * half

        for cp in local_copies:
            cp.wait()

    return pl.pallas_call(
        body,
        out_shape=jax.ShapeDtypeStruct((m_out, n), x.dtype),
        in_specs=[pl.BlockSpec(memory_space=pl.ANY)],
        out_specs=pl.BlockSpec(memory_space=pl.ANY),
        scratch_shapes=[
            pltpu.SemaphoreType.DMA((NUM_CHUNKS,)),
            pltpu.SemaphoreType.DMA((NUM_CHUNKS,)),
            pltpu.SemaphoreType.DMA((NUM_CHUNKS,)),
            pltpu.SemaphoreType.DMA((NUM_CHUNKS,)),
            pltpu.SemaphoreType.DMA((NUM_CHUNKS,)),
        ],
        compiler_params=pltpu.CompilerParams(collective_id=0),
    )(x)


# device time: 497111 ns/iter; 4.2810x vs baseline; 4.2810x over previous
import jax
import jax.numpy as jnp
from jax import lax
from jax.experimental import pallas as pl
from jax.experimental.pallas import tpu as pltpu

NUM_CHUNKS = 32
L_CHUNKS = 16
L_BUFS = 4


def kernel(x):
    m_per, n = x.shape
    m_out = 2 * m_per
    half = m_per // 2
    chunk = half // NUM_CHUNKS
    lchunk = m_per // L_CHUNKS

    def body(x_ref, out_ref, stage_ref, rd_sems, wr_sems,
             x_send_sems, x_recv_sems, y_send_sems, y_recv_sems):
        my_x = lax.axis_index("x")
        my_y = lax.axis_index("y")
        x_nbr = (1 - my_x, my_y)
        y_nbr = (my_x, 1 - my_y)

        barrier_sem = pltpu.get_barrier_semaphore()
        for nbr in (x_nbr, y_nbr):
            pl.semaphore_signal(
                barrier_sem, inc=1,
                device_id=nbr, device_id_type=pl.DeviceIdType.MESH,
            )
        pl.semaphore_wait(barrier_sem, 2)

        send_base = my_x * m_per + my_y * half
        recv_base = (1 - my_x) * m_per + my_y * half

        x_rdmas = []
        for c in range(NUM_CHUNKS):
            r = pltpu.make_async_remote_copy(
                src_ref=x_ref.at[pl.ds(my_y * half + c * chunk, chunk), :],
                dst_ref=out_ref.at[pl.ds(send_base + c * chunk, chunk), :],
                send_sem=x_send_sems.at[c],
                recv_sem=x_recv_sems.at[c],
                device_id=x_nbr,
                device_id_type=pl.DeviceIdType.MESH,
            )
            r.start()
            x_rdmas.append(r)

        reads = [None] * L_CHUNKS
        writes = [None] * L_CHUNKS
        for c in range(L_BUFS):
            reads[c] = pltpu.make_async_copy(
                x_ref.at[pl.ds(c * lchunk, lchunk), :],
                stage_ref.at[c], rd_sems.at[c])
            reads[c].start()
        for c in range(L_CHUNKS):
            b = c % L_BUFS
            reads[c].wait()
            writes[c] = pltpu.make_async_copy(
                stage_ref.at[b],
                out_ref.at[pl.ds(my_x * m_per + c * lchunk, lchunk), :],
                wr_sems.at[b])
            writes[c].start()
            nxt = c + L_BUFS
            if nxt < L_CHUNKS:
                writes[c].wait()
                reads[nxt] = pltpu.make_async_copy(
                    x_ref.at[pl.ds(nxt * lchunk, lchunk), :],
                    stage_ref.at[b], rd_sems.at[b])
                reads[nxt].start()
        for c in range(L_CHUNKS - L_BUFS, L_CHUNKS):
            writes[c].wait()

        y_rdmas = []
        for c in range(NUM_CHUNKS):
            x_rdmas[c].wait_recv()
            r = pltpu.make_async_remote_copy(
                src_ref=out_ref.at[pl.ds(recv_base + c * chunk, chunk), :],
                dst_ref=out_ref.at[pl.ds(recv_base + c * chunk, chunk), :],
                send_sem=y_send_sems.at[c],
                recv_sem=y_recv_sems.at[c],
                device_id=y_nbr,
                device_id_type=pl.DeviceIdType.MESH,
            )
            r.start()
            y_rdmas.append(r)

        for c in range(NUM_CHUNKS):
            y_rdmas[c].wait_recv()
        for c in range(NUM_CHUNKS):
            x_rdmas[c].wait_send()
            y_rdmas[c].wait_send()

    return pl.pallas_call(
        body,
        out_shape=jax.ShapeDtypeStruct((m_out, n), x.dtype),
        in_specs=[pl.BlockSpec(memory_space=pl.ANY)],
        out_specs=pl.BlockSpec(memory_space=pl.ANY),
        scratch_shapes=[
            pltpu.VMEM((L_BUFS, m_per // L_CHUNKS, n), x.dtype),
            pltpu.SemaphoreType.DMA((L_BUFS,)),
            pltpu.SemaphoreType.DMA((L_BUFS,)),
            pltpu.SemaphoreType.DMA((NUM_CHUNKS,)),
            pltpu.SemaphoreType.DMA((NUM_CHUNKS,)),
            pltpu.SemaphoreType.DMA((NUM_CHUNKS,)),
            pltpu.SemaphoreType.DMA((NUM_CHUNKS,)),
        ],
        compiler_params=pltpu.CompilerParams(collective_id=0),
    )(x)


# device time: 465936 ns/iter; 4.5674x vs baseline; 1.0669x over previous
import jax
import jax.numpy as jnp
from jax import lax
from jax.experimental import pallas as pl
from jax.experimental.pallas import tpu as pltpu

NUM_CHUNKS = 32
L_CHUNKS = 16
L_BUFS = 4


def kernel(x):
    m_per, n = x.shape
    m_out = 2 * m_per
    half = m_per // 2
    chunk = half // NUM_CHUNKS
    lchunk = m_per // L_CHUNKS

    def body(x_ref, out_ref, stage_ref, rd_sems, wr_sems,
             x_send_sems, x_recv_sems, y_send_sems, y_recv_sems):
        my_x = lax.axis_index("x")
        my_y = lax.axis_index("y")
        x_nbr = (1 - my_x, my_y)
        y_nbr = (my_x, 1 - my_y)

        barrier_sem = pltpu.get_barrier_semaphore()
        for nbr in (x_nbr, y_nbr):
            pl.semaphore_signal(
                barrier_sem, inc=1,
                device_id=nbr, device_id_type=pl.DeviceIdType.MESH,
            )
        pl.semaphore_wait(barrier_sem, 2)

        send_base = my_x * m_per + my_y * half
        recv_base = (1 - my_x) * m_per + my_y * half

        x_rdmas = []
        for c in range(NUM_CHUNKS):
            r = pltpu.make_async_remote_copy(
                src_ref=x_ref.at[pl.ds(my_y * half + c * chunk, chunk), :],
                dst_ref=out_ref.at[pl.ds(send_base + c * chunk, chunk), :],
                send_sem=x_send_sems.at[c],
                recv_sem=x_recv_sems.at[c],
                device_id=x_nbr,
                device_id_type=pl.DeviceIdType.MESH,
            )
            r.start()
            x_rdmas.append(r)

        reads = [None] * L_CHUNKS
        writes = [None] * L_CHUNKS
        for c in range(L_BUFS):
            reads[c] = pltpu.make_async_copy(
                x_ref.at[pl.ds(c * lchunk, lchunk), :],
                stage_ref.at[c], rd_sems.at[c])
            reads[c].start()

        def advance_local(k):
            b = k % L_BUFS
            reads[k].wait()
            writes[k] = pltpu.make_async_copy(
                stage_ref.at[b],
                out_ref.at[pl.ds(my_x * m_per + k * lchunk, lchunk), :],
                wr_sems.at[b])
            writes[k].start()
            nxt = k + L_BUFS
            if nxt < L_CHUNKS:
                writes[k].wait()
                reads[nxt] = pltpu.make_async_copy(
                    x_ref.at[pl.ds(nxt * lchunk, lchunk), :],
                    stage_ref.at[b], rd_sems.at[b])
                reads[nxt].start()

        y_rdmas = []
        for c in range(NUM_CHUNKS):
            x_rdmas[c].wait_recv()
            r = pltpu.make_async_remote_copy(
                src_ref=out_ref.at[pl.ds(recv_base + c * chunk, chunk), :],
                dst_ref=out_ref.at[pl.ds(recv_base + c * chunk, chunk), :],
                send_sem=y_send_sems.at[c],
                recv_sem=y_recv_sems.at[c],
                device_id=y_nbr,
                device_id_type=pl.DeviceIdType.MESH,
            )
            r.start()
            y_rdmas.append(r)
            if c % 2 == 0 and c // 2 < L_CHUNKS:
                advance_local(c // 2)
        for c in range(L_CHUNKS - L_BUFS, L_CHUNKS):
            writes[c].wait()

        for c in range(NUM_CHUNKS):
            y_rdmas[c].wait_recv()
        for c in range(NUM_CHUNKS):
            x_rdmas[c].wait_send()
            y_rdmas[c].wait_send()

    return pl.pallas_call(
        body,
        out_shape=jax.ShapeDtypeStruct((m_out, n), x.dtype),
        in_specs=[pl.BlockSpec(memory_space=pl.ANY)],
        out_specs=pl.BlockSpec(memory_space=pl.ANY),
        scratch_shapes=[
            pltpu.VMEM((L_BUFS, m_per // L_CHUNKS, n), x.dtype),
            pltpu.SemaphoreType.DMA((L_BUFS,)),
            pltpu.SemaphoreType.DMA((L_BUFS,)),
            pltpu.SemaphoreType.DMA((NUM_CHUNKS,)),
            pltpu.SemaphoreType.DMA((NUM_CHUNKS,)),
            pltpu.SemaphoreType.DMA((NUM_CHUNKS,)),
            pltpu.SemaphoreType.DMA((NUM_CHUNKS,)),
        ],
        compiler_params=pltpu.CompilerParams(collective_id=0),
    )(x)


# device time: 461617 ns/iter; 4.6101x vs baseline; 1.0094x over previous
import jax
import jax.numpy as jnp
from jax import lax
from jax.experimental import pallas as pl
from jax.experimental.pallas import tpu as pltpu

NUM_CHUNKS = 64
L_CHUNKS = 16
L_BUFS = 4


def kernel(x):
    m_per, n = x.shape
    m_out = 2 * m_per
    half = m_per // 2
    chunk = half // NUM_CHUNKS
    lchunk = m_per // L_CHUNKS

    def body(x_ref, out_ref, stage_ref, rd_sems, wr_sems,
             x_send_sems, x_recv_sems, y_send_sems, y_recv_sems):
        my_x = lax.axis_index("x")
        my_y = lax.axis_index("y")
        x_nbr = (1 - my_x, my_y)
        y_nbr = (my_x, 1 - my_y)

        barrier_sem = pltpu.get_barrier_semaphore()
        for nbr in (x_nbr, y_nbr):
            pl.semaphore_signal(
                barrier_sem, inc=1,
                device_id=nbr, device_id_type=pl.DeviceIdType.MESH,
            )
        pl.semaphore_wait(barrier_sem, 2)

        send_base = my_x * m_per + my_y * half
        recv_base = (1 - my_x) * m_per + my_y * half

        x_rdmas = []
        for c in range(NUM_CHUNKS):
            r = pltpu.make_async_remote_copy(
                src_ref=x_ref.at[pl.ds(my_y * half + c * chunk, chunk), :],
                dst_ref=out_ref.at[pl.ds(send_base + c * chunk, chunk), :],
                send_sem=x_send_sems.at[c],
                recv_sem=x_recv_sems.at[c],
                device_id=x_nbr,
                device_id_type=pl.DeviceIdType.MESH,
            )
            r.start()
            x_rdmas.append(r)

        reads = [None] * L_CHUNKS
        writes = [None] * L_CHUNKS
        for c in range(L_BUFS):
            reads[c] = pltpu.make_async_copy(
                x_ref.at[pl.ds(c * lchunk, lchunk), :],
                stage_ref.at[c], rd_sems.at[c])
            reads[c].start()

        def advance_local(k):
            b = k % L_BUFS
            reads[k].wait()
            writes[k] = pltpu.make_async_copy(
                stage_ref.at[b],
                out_ref.at[pl.ds(my_x * m_per + k * lchunk, lchunk), :],
                wr_sems.at[b])
            writes[k].start()
            nxt = k + L_BUFS
            if nxt < L_CHUNKS:
                writes[k].wait()
                reads[nxt] = pltpu.make_async_copy(
                    x_ref.at[pl.ds(nxt * lchunk, lchunk), :],
                    stage_ref.at[b], rd_sems.at[b])
                reads[nxt].start()

        y_rdmas = []
        for c in range(NUM_CHUNKS):
            x_rdmas[c].wait_recv()
            r = pltpu.make_async_remote_copy(
                src_ref=out_ref.at[pl.ds(recv_base + c * chunk, chunk), :],
                dst_ref=out_ref.at[pl.ds(recv_base + c * chunk, chunk), :],
                send_sem=y_send_sems.at[c],
                recv_sem=y_recv_sems.at[c],
                device_id=y_nbr,
                device_id_type=pl.DeviceIdType.MESH,
            )
            r.start()
            y_rdmas.append(r)
            if c % 4 == 0 and c // 4 < L_CHUNKS:
                advance_local(c // 4)
        for c in range(L_CHUNKS - L_BUFS, L_CHUNKS):
            writes[c].wait()

        for c in range(NUM_CHUNKS):
            y_rdmas[c].wait_recv()
        for c in range(NUM_CHUNKS):
            x_rdmas[c].wait_send()
            y_rdmas[c].wait_send()

    return pl.pallas_call(
        body,
        out_shape=jax.ShapeDtypeStruct((m_out, n), x.dtype),
        in_specs=[pl.BlockSpec(memory_space=pl.ANY)],
        out_specs=pl.BlockSpec(memory_space=pl.ANY),
        scratch_shapes=[
            pltpu.VMEM((L_BUFS, m_per // L_CHUNKS, n), x.dtype),
            pltpu.SemaphoreType.DMA((L_BUFS,)),
            pltpu.SemaphoreType.DMA((L_BUFS,)),
            pltpu.SemaphoreType.DMA((NUM_CHUNKS,)),
            pltpu.SemaphoreType.DMA((NUM_CHUNKS,)),
            pltpu.SemaphoreType.DMA((NUM_CHUNKS,)),
            pltpu.SemaphoreType.DMA((NUM_CHUNKS,)),
        ],
        compiler_params=pltpu.CompilerParams(collective_id=0),
    )(x)
